# Initial kernel scaffold; baseline (speedup 1.0000x reference)
import jax
import jax.numpy as jnp
from jax.experimental import pallas as pl


def kernel(inputs, labels, sample_ids, weight):
    raise NotImplementedError("write your pallas kernel here")


# speedup vs baseline: 2.7345x; 2.7345x over previous
import functools

import jax
import jax.numpy as jnp
from jax import lax
from jax.experimental import pallas as pl
from jax.experimental.pallas import tpu as pltpu
from jax.experimental.pallas import tpu_sc as plsc

NTOK = 100000
NSAMP = 200
NHID = 64
B = 128

BL = 32768
NW = 32
IPW = B // NW
C0, C1 = 104, 96
UNROLL = 4


def _exp_mm_body(wt_ref, x_ref, e_ref):
    e_ref[...] = jnp.exp(
        lax.dot_general(
            wt_ref[...], x_ref[...],
            dimension_numbers=(((0,), (1,)), ((), ())),
            preferred_element_type=jnp.float32,
        )
    )


def _finish_body(seg_ref, elab_ref, o_ref):
    o_ref[...] = jnp.sum(
        jnp.log(seg_ref[...]) - jnp.log(elab_ref[...]), axis=0, keepdims=True
    )


@functools.cache
def _sc_gather_sum():
    mesh = plsc.VectorSubcoreMesh(core_axis_name="c", subcore_axis_name="s")

    @functools.partial(
        pl.kernel,
        mesh=mesh,
        out_type=[
            jax.ShapeDtypeStruct((B, B), jnp.float32),
            jax.ShapeDtypeStruct((B, B), jnp.float32),
        ],
        scratch_types=[
            pltpu.VMEM((IPW * NSAMP,), jnp.int32),
            pltpu.VMEM((NSAMP, B), jnp.float32),
            pltpu.VMEM((NSAMP, B), jnp.float32),
            pltpu.VMEM((2 * IPW,), jnp.int32),
            pltpu.VMEM((2 * IPW, B), jnp.float32),
            pltpu.VMEM((IPW, B), jnp.float32),
            pltpu.SemaphoreType.DMA,
            pltpu.SemaphoreType.DMA,
            pltpu.SemaphoreType.DMA,
        ],
    )
    def body(samp_hbm, lab_hbm, e_hbm, seg_hbm, elab_hbm,
             idx_v, buf_a, buf_b, labidx_v, labrows_v, seg_v,
             sem_a, sem_b, sem_l):
        wid = lax.axis_index("s") * 2 + lax.axis_index("c")
        base = wid * IPW

        pltpu.sync_copy(samp_hbm.at[pl.ds(base * NSAMP, IPW * NSAMP)], idx_v)

        bufs = [buf_a, buf_b]
        sems = [sem_a, sem_b]
        cps = [None, None]

        def fire(li):
            b = bufs[li % 2]
            s = sems[li % 2]
            off = li * NSAMP
            c1 = pltpu.async_copy(
                e_hbm.at[idx_v.at[pl.ds(off, C0)]], b.at[pl.ds(0, C0)], s)
            c2 = pltpu.async_copy(
                e_hbm.at[idx_v.at[pl.ds(off + C0, C1)]], b.at[pl.ds(C0, C1)], s)
            cps[li % 2] = (c1, c2)

        def sum_rows(b):
            def step(k, acc):
                r = k * UNROLL
                out = []
                for v in range(B // 16):
                    a = acc[v]
                    for u in range(UNROLL):
                        a = a + b[r + u, pl.ds(v * 16, 16)]
                    out.append(a)
                return tuple(out)

            init = tuple(jnp.zeros((16,), jnp.float32) for _ in range(B // 16))
            return lax.fori_loop(0, NSAMP // UNROLL, step, init)

        even = (wid % 2) == 0
        lbase = (wid // 2) * (2 * IPW)

        fire(0)

        @pl.when(even)
        def _():
            pltpu.sync_copy(lab_hbm.at[pl.ds(lbase, 2 * IPW)], labidx_v)
            pltpu.async_copy(e_hbm.at[labidx_v], labrows_v, sem_l)

        for li in range(IPW):
            if li + 1 < IPW:
                fire(li + 1)
            for c in cps[li % 2]:
                c.wait()
            acc = sum_rows(bufs[li % 2])
            for v in range(B // 16):
                seg_v[li, pl.ds(v * 16, 16)] = acc[v]

        pltpu.sync_copy(seg_v, seg_hbm.at[pl.ds(base, IPW)])

        @pl.when(even)
        def _():
            pltpu.make_async_copy(e_hbm.at[labidx_v], labrows_v, sem_l).wait()
            pltpu.sync_copy(labrows_v, elab_hbm.at[pl.ds(lbase, 2 * IPW)])

    return body


def kernel(inputs, labels, sample_ids, weight):
    samp = sample_ids.astype(jnp.int32).reshape(-1)
    lab = labels.astype(jnp.int32)

    e_all = pl.pallas_call(
        _exp_mm_body,
        grid=(pl.cdiv(NTOK, BL),),
        in_specs=[
            pl.BlockSpec((NHID, BL), lambda i: (0, i)),
            pl.BlockSpec((B, NHID), lambda i: (0, 0)),
        ],
        out_specs=pl.BlockSpec((BL, B), lambda i: (i, 0)),
        out_shape=jax.ShapeDtypeStruct((NTOK, B), jnp.float32),
    )(weight.T, inputs)

    seg, elab = _sc_gather_sum()(samp, lab, e_all)

    out = pl.pallas_call(
        _finish_body,
        out_shape=jax.ShapeDtypeStruct((1, B), jnp.float32),
    )(seg, elab)
    return out[0]


# speedup vs baseline: 2.9033x; 1.0617x over previous
import functools

import jax
import jax.numpy as jnp
from jax import lax
from jax.experimental import pallas as pl
from jax.experimental.pallas import tpu as pltpu
from jax.experimental.pallas import tpu_sc as plsc

NTOK = 100000
NSAMP = 200
NHID = 64
B = 128

SPLIT = 50176
BLP = 12544
NW = 32
IPW = B // NW
C0, C1 = 104, 96
UNROLL = 4
NG = B // 16
LPL = 16


def _exp_mm_body(wt1_ref, wt2_ref, x_ref, e_ref):
    dn = (((0,), (1,)), ((), ()))
    e1 = jnp.exp(lax.dot_general(wt1_ref[...], x_ref[...], dimension_numbers=dn,
                                 preferred_element_type=jnp.float32))
    e2 = jnp.exp(lax.dot_general(wt2_ref[...], x_ref[...], dimension_numbers=dn,
                                 preferred_element_type=jnp.float32))
    lo = lax.bitcast_convert_type(e1.astype(jnp.bfloat16), jnp.uint16)
    hi = lax.bitcast_convert_type(e2.astype(jnp.bfloat16), jnp.uint16)
    packed = lo.astype(jnp.uint32) | (hi.astype(jnp.uint32) << 16)
    e_ref[...] = lax.bitcast_convert_type(packed, jnp.float32)


def _finish_body(seg_ref, elab_ref, o_ref):
    o_ref[...] = jnp.sum(
        jnp.log(seg_ref[...]) - jnp.log(elab_ref[...]), axis=0, keepdims=True
    )


@functools.cache
def _sc_gather_sum():
    mesh = plsc.VectorSubcoreMesh(core_axis_name="c", subcore_axis_name="s")

    @functools.partial(
        pl.kernel,
        mesh=mesh,
        out_type=[
            jax.ShapeDtypeStruct((B, B), jnp.float32),
            jax.ShapeDtypeStruct((B, B), jnp.float32),
        ],
        scratch_types=[
            pltpu.VMEM((IPW * NSAMP,), jnp.int32),
            pltpu.VMEM((IPW * NSAMP,), jnp.int32),
            pltpu.VMEM((NSAMP, B), jnp.float32),
            pltpu.VMEM((NSAMP, B), jnp.float32),
            pltpu.VMEM((LPL,), jnp.int32),
            pltpu.VMEM((LPL,), jnp.int32),
            pltpu.VMEM((LPL, B), jnp.float32),
            pltpu.VMEM((LPL, B), jnp.float32),
            pltpu.VMEM((IPW, B), jnp.float32),
            pltpu.SemaphoreType.DMA,
            pltpu.SemaphoreType.DMA,
            pltpu.SemaphoreType.DMA,
        ],
        compiler_params=pltpu.CompilerParams(needs_layout_passes=False),
    )
    def body(samp_hbm, lab_hbm, e_hbm, seg_hbm, elab_hbm,
             idx_v, hs_v, buf_a, buf_b, labidx_v, hslab_v, labrows_v,
             labunp_v, seg_v, sem_a, sem_b, sem_l):
        wid = lax.axis_index("s") * 2 + lax.axis_index("c")
        base = wid * IPW

        pltpu.sync_copy(samp_hbm.at[pl.ds(base * NSAMP, IPW * NSAMP)], idx_v)

        def remap(idx_ref, hs_ref, off):
            v = idx_ref[pl.ds(off, 16)]
            m = v >= SPLIT
            idx_ref[pl.ds(off, 16)] = v - jnp.where(m, SPLIT, 0)
            hs_ref[pl.ds(off, 16)] = jnp.where(m, 0, 16)

        for g in range(IPW * NSAMP // 16):
            remap(idx_v, hs_v, g * 16)

        bufs = [buf_a, buf_b]
        sems = [sem_a, sem_b]
        cps = [None, None]

        def fire(li):
            b = bufs[li % 2]
            s = sems[li % 2]
            off = li * NSAMP
            c1 = pltpu.async_copy(
                e_hbm.at[idx_v.at[pl.ds(off, C0)]], b.at[pl.ds(0, C0)], s)
            c2 = pltpu.async_copy(
                e_hbm.at[idx_v.at[pl.ds(off + C0, C1)]], b.at[pl.ds(C0, C1)], s)
            cps[li % 2] = (c1, c2)

        def unpack_add(b, r, shift_ref, sbase, acc):
            s = plsc.load_gather(
                shift_ref, [jnp.broadcast_to(sbase + r, (16,))])
            out = []
            for v in range(NG):
                w = plsc.bitcast(b[r, pl.ds(v * 16, 16)], jnp.int32)
                val = plsc.bitcast((w << s) & jnp.int32(-65536), jnp.float32)
                out.append(acc[v] + val)
            return out

        def sum_rows(b, li):
            def step(k, acc):
                r = k * UNROLL
                cur = list(acc)
                for u in range(UNROLL):
                    cur = unpack_add(b, r + u, hs_v, li * NSAMP, cur)
                return tuple(cur)

            init = tuple(jnp.zeros((16,), jnp.float32) for _ in range(NG))
            return lax.fori_loop(0, NSAMP // UNROLL, step, init)

        is_lab_worker = (wid % 4) == 0
        lbase = (wid // 4) * LPL

        fire(0)

        @pl.when(is_lab_worker)
        def _():
            pltpu.sync_copy(lab_hbm.at[pl.ds(lbase, LPL)], labidx_v)
            lv = labidx_v[pl.ds(0, 16)]
            m = lv >= SPLIT
            labidx_v[pl.ds(0, 16)] = lv - jnp.where(m, SPLIT, 0)
            hslab_v[pl.ds(0, 16)] = jnp.where(m, 0, 16)
            pltpu.async_copy(e_hbm.at[labidx_v], labrows_v, sem_l)

        for li in range(IPW):
            if li + 1 < IPW:
                fire(li + 1)
            for c in cps[li % 2]:
                c.wait()
            acc = sum_rows(bufs[li % 2], li)
            for v in range(NG):
                seg_v[li, pl.ds(16 * v, 16)] = acc[v]

        pltpu.sync_copy(seg_v, seg_hbm.at[pl.ds(base, IPW)])

        @pl.when(is_lab_worker)
        def _():
            pltpu.make_async_copy(e_hbm.at[labidx_v], labrows_v, sem_l).wait()
            for lr in range(LPL):
                s = plsc.load_gather(
                    hslab_v, [jnp.broadcast_to(lr, (16,))])
                for v in range(NG):
                    w = plsc.bitcast(labrows_v[lr, pl.ds(v * 16, 16)], jnp.int32)
                    labunp_v[lr, pl.ds(v * 16, 16)] = plsc.bitcast(
                        (w << s) & jnp.int32(-65536), jnp.float32)
            pltpu.sync_copy(labunp_v, elab_hbm.at[pl.ds(lbase, LPL)])

    return body


def kernel(inputs, labels, sample_ids, weight):
    samp = sample_ids.astype(jnp.int32).reshape(-1)
    lab = labels.astype(jnp.int32)

    e_all = pl.pallas_call(
        _exp_mm_body,
        grid=(SPLIT // BLP,),
        in_specs=[
            pl.BlockSpec((NHID, BLP), lambda i: (0, i)),
            pl.BlockSpec((NHID, BLP), lambda i: (0, i + SPLIT // BLP)),
            pl.BlockSpec((B, NHID), lambda i: (0, 0)),
        ],
        out_specs=pl.BlockSpec((BLP, B), lambda i: (i, 0)),
        out_shape=jax.ShapeDtypeStruct((SPLIT, B), jnp.float32),
    )(weight.T, weight.T, inputs)

    seg, elab = _sc_gather_sum()(samp, lab, e_all)

    out = pl.pallas_call(
        _finish_body,
        out_shape=jax.ShapeDtypeStruct((1, B), jnp.float32),
    )(seg, elab)
    return out[0]
